# baseline (speedup 1.0000x reference)
import functools

import jax
import jax.numpy as jnp
from jax import lax
from jax.experimental import pallas as pl
from jax.experimental.pallas import tpu as pltpu
from jax.experimental.pallas import tpu_sc as plsc

_NC = 2
_NS = 16
_NW = _NC * _NS
_L = 16


def _worker_id():
    return lax.axis_index("s") * _NC + lax.axis_index("c")


def _mesh():
    return plsc.VectorSubcoreMesh(core_axis_name="c", subcore_axis_name="s",
                                  num_cores=_NC, num_subcores=_NS)


def _make_pack(v, d):
    assert d == 64 and v % 2 == 0
    n_full_sr = v // 128
    tail = v - n_full_sr * 128
    assert tail in (0, 64)
    n_blk = n_full_sr // 2
    assert n_blk * 2 == n_full_sr
    per, extra = divmod(n_blk, _NW)

    @functools.partial(
        pl.kernel,
        mesh=_mesh(),
        out_type=jax.ShapeDtypeStruct((v // 2, 128), jnp.float32),
        compiler_params=pltpu.CompilerParams(use_tc_tiling_on_sc=True,
                                             needs_layout_passes=False),
        scratch_types=[
            pltpu.VMEM((d, 256), jnp.float32),
            pltpu.VMEM((d, 256), jnp.float32),
            pltpu.VMEM((d, 64), jnp.float32),
            pltpu.VMEM((128, 128), jnp.float32),
            pltpu.VMEM((128, 128), jnp.float32),
            pltpu.SemaphoreType.DMA,
            pltpu.SemaphoreType.DMA,
            pltpu.SemaphoreType.DMA,
            pltpu.SemaphoreType.DMA,
        ],
    )
    def pack_kernel(tt_hbm, tail_hbm, out_hbm, in0, in1, in_tail,
                    o0, o1, gi0, gi1, so0, so1):
        bin_ = [in0, in1]
        bout = [o0, o1]
        isem = [gi0, gi1]
        osem = [so0, so1]
        w = _worker_id()
        base = per * w + jnp.minimum(w, extra)
        cnt = per + jnp.where(w < extra, 1, 0)
        iota = lax.iota(jnp.int32, _L)

        def fire_load(t, slot):
            pltpu.async_copy(tt_hbm.at[:, pl.ds(t * 256, 256)], bin_[slot],
                             isem[slot])

        def wait_load(t, slot):
            pltpu.make_async_copy(tt_hbm.at[:, pl.ds(t * 256, 256)],
                                  bin_[slot], isem[slot]).wait()

        def fire_store(t, slot):
            pltpu.async_copy(bout[slot], out_hbm.at[pl.ds(t * 128, 128), :],
                             osem[slot])

        def wait_store(t, slot):
            pltpu.make_async_copy(bout[slot],
                                  out_hbm.at[pl.ds(t * 128, 128), :],
                                  osem[slot]).wait()

        def transpose_block(src, dst, n_rows):
            @pl.loop(0, n_rows)
            def _(r):
                for cg in range(8):
                    c0 = cg * 16
                    ilvec = jnp.full((_L,), 0, jnp.int32) + (
                        2 * r + (1 if c0 >= 64 else 0))
                    jvec = (c0 % 64) + iota
                    vals = plsc.load_gather(src, [jvec, ilvec])
                    dst[r, pl.ds(c0, 16)] = vals

        @pl.when(cnt >= 1)
        def _():
            fire_load(base, 0)

        @pl.loop(0, cnt)
        def _(k):
            for s in (0, 1):
                @pl.when((k & 1) == s)
                def _(s=s):
                    t = base + k
                    o = 1 - s

                    @pl.when(k + 1 < cnt)
                    def _():
                        fire_load(t + 1, o)

                    wait_load(t, s)

                    @pl.when(k >= 2)
                    def _():
                        wait_store(t - 2, s)

                    transpose_block(bin_[s], bout[s], 128)
                    fire_store(t, s)

        for s in (0, 1):
            @pl.when((cnt >= 2) & (((cnt - 2) & 1) == s))
            def _(s=s):
                wait_store(base + cnt - 2, s)

            @pl.when((cnt >= 1) & (((cnt - 1) & 1) == s))
            def _(s=s):
                wait_store(base + cnt - 1, s)

        if tail:
            @pl.when(w == _NW - 1)
            def _():
                pltpu.sync_copy(tail_hbm, in_tail)
                transpose_block(in_tail, bout[0], 32)
                pltpu.sync_copy(bout[0].at[pl.ds(0, 32), :],
                                out_hbm.at[pl.ds(n_full_sr * 64, 32), :])

    return pack_kernel


def _make_gather(v, d, hist, batch, blk):
    bpw = batch // _NW
    nbi = bpw // blk
    assert nbi * blk == bpw and nbi % 2 == 0

    @functools.partial(
        pl.kernel,
        mesh=_mesh(),
        out_type=jax.ShapeDtypeStruct((hist, d, batch), jnp.float32),
        compiler_params=pltpu.CompilerParams(use_tc_tiling_on_sc=True,
                                             needs_layout_passes=False),
        scratch_types=[
            pltpu.VMEM((hist, bpw), jnp.int32),
            pltpu.VMEM((blk,), jnp.int32),
            pltpu.VMEM((blk,), jnp.int32),
            pltpu.VMEM((blk,), jnp.int32),
            pltpu.VMEM((blk,), jnp.int32),
            pltpu.VMEM((blk, 128), jnp.float32),
            pltpu.VMEM((blk, 128), jnp.float32),
            pltpu.VMEM((d, blk), jnp.float32),
            pltpu.VMEM((d, blk), jnp.float32),
            pltpu.SemaphoreType.DMA,
            pltpu.SemaphoreType.DMA,
            pltpu.SemaphoreType.DMA,
            pltpu.SemaphoreType.DMA,
        ],
    )
    def gather_kernel(tab_hbm, xt_hbm, out_hbm, xbuf, ri0, ri1, pa0, pa1,
                      r0, r1, t0, t1, g0, g1, so0, so1):
        rowidx = [ri0, ri1]
        par64 = [pa0, pa1]
        rows = [r0, r1]
        tbuf = [t0, t1]
        gsem = [g0, g1]
        osem = [so0, so1]
        w = _worker_id()
        b_base = w * bpw
        iota = lax.iota(jnp.int32, _L)

        pltpu.sync_copy(xt_hbm.at[:, pl.ds(b_base, bpw)], xbuf)

        def prep_idx(h, bi, slot):
            @pl.loop(0, blk // _L)
            def _(g):
                raw = xbuf[h, pl.ds(bi * blk + g * _L, _L)]
                par64[slot][pl.ds(g * _L, _L)] = (raw & 1) * 64
                rowidx[slot][pl.ds(g * _L, _L)] = raw >> 1

        def fire_gather(slot):
            pltpu.async_copy(tab_hbm.at[rowidx[slot]], rows[slot],
                             gsem[slot])

        def wait_gather(slot):
            pltpu.make_async_copy(tab_hbm.at[rowidx[slot]], rows[slot],
                                  gsem[slot]).wait()

        def fire_out(h, bi, slot):
            b0 = b_base + bi * blk
            pltpu.async_copy(tbuf[slot],
                             out_hbm.at[h, :, pl.ds(b0, blk)], osem[slot])

        def wait_out(h, bi, slot):
            b0 = b_base + bi * blk
            pltpu.make_async_copy(tbuf[slot],
                                  out_hbm.at[h, :, pl.ds(b0, blk)],
                                  osem[slot]).wait()

        def transpose_unit(slot):
            @pl.loop(0, blk // _L)
            def _(g):
                bvec = g * _L + iota
                pvec = par64[slot][pl.ds(g * _L, _L)]
                for j in range(d):
                    vals = plsc.load_gather(rows[slot], [bvec, pvec + j])
                    tbuf[slot][j, pl.ds(g * _L, _L)] = vals

        prep_idx(0, 0, 0)
        fire_gather(0)

        @pl.loop(0, hist)
        def _(h):
            for bi in range(nbi):
                s = bi & 1
                o = 1 - s
                if bi + 1 < nbi:
                    prep_idx(h, bi + 1, o)
                    fire_gather(o)
                else:
                    @pl.when(h + 1 < hist)
                    def _():
                        prep_idx(h + 1, 0, o)
                        fire_gather(o)

                wait_gather(s)

                if bi >= 2:
                    wait_out(h, bi - 2, s)
                else:
                    @pl.when(h >= 1)
                    def _(bi=bi):
                        wait_out(h - 1, nbi - 2 + bi, s)

                transpose_unit(s)
                fire_out(h, bi, s)

        wait_out(hist - 1, nbi - 2, 0)
        wait_out(hist - 1, nbi - 1, 1)

    return gather_kernel


@jax.jit
def kernel(x, table):
    batch, hist = x.shape
    vocab, dim = table.shape
    xt = x.T
    tt = table.T
    n_full = (vocab // 128) * 128
    ttail = tt[:, n_full:]
    packed = _make_pack(vocab, dim)(tt, ttail)
    outt = _make_gather(vocab, dim, hist, batch, 128)(packed, xt)
    return jnp.transpose(outt, (2, 0, 1))


# speedup vs baseline: 2.2967x; 2.2967x over previous
import functools

import jax
import jax.numpy as jnp
from jax import lax
from jax.experimental import pallas as pl
from jax.experimental.pallas import tpu as pltpu
from jax.experimental.pallas import tpu_sc as plsc

_NC = 2
_NS = 16
_NW = _NC * _NS


def _make_gather(v, d, hist, batch):
    bpw = batch // _NW
    assert bpw * _NW == batch and hist % 2 == 0
    mesh = plsc.VectorSubcoreMesh(core_axis_name="c", subcore_axis_name="s",
                                  num_cores=_NC, num_subcores=_NS)

    @functools.partial(
        pl.kernel,
        mesh=mesh,
        out_type=jax.ShapeDtypeStruct((hist, batch, d), jnp.float32),
        compiler_params=pltpu.CompilerParams(use_tc_tiling_on_sc=False),
        scratch_types=[
            pltpu.VMEM((bpw,), jnp.int32),
            pltpu.VMEM((bpw,), jnp.int32),
            pltpu.VMEM((bpw, d), jnp.float32),
            pltpu.VMEM((bpw, d), jnp.float32),
            pltpu.SemaphoreType.DMA,
            pltpu.SemaphoreType.DMA,
            pltpu.SemaphoreType.DMA,
            pltpu.SemaphoreType.DMA,
        ],
    )
    def gather_kernel(tab_hbm, xt_hbm, out_hbm, i0, i1, r0, r1,
                      g0, g1, o0, o1):
        idx_v = [i0, i1]
        rows = [r0, r1]
        gsem = [g0, g1]
        osem = [o0, o1]
        w = lax.axis_index("s") * _NC + lax.axis_index("c")
        b_base = w * bpw

        def prep(h, slot):
            pltpu.sync_copy(xt_hbm.at[h, pl.ds(b_base, bpw)], idx_v[slot])

        def fire_gather(slot):
            pltpu.async_copy(tab_hbm.at[idx_v[slot]], rows[slot],
                             gsem[slot])

        def wait_gather(slot):
            pltpu.make_async_copy(tab_hbm.at[idx_v[slot]], rows[slot],
                                  gsem[slot]).wait()

        def fire_out(h, slot):
            pltpu.async_copy(rows[slot],
                             out_hbm.at[h, pl.ds(b_base, bpw), :],
                             osem[slot])

        def wait_out(h, slot):
            pltpu.make_async_copy(rows[slot],
                                  out_hbm.at[h, pl.ds(b_base, bpw), :],
                                  osem[slot]).wait()

        prep(0, 0)
        fire_gather(0)

        @pl.loop(0, hist // 2)
        def _(hh):
            for p in (0, 1):
                h = 2 * hh + p
                s = p
                o = 1 - p

                @pl.when(h + 1 < hist)
                def _(h=h, s=s, o=o):
                    prep(h + 1, o)

                    @pl.when(h >= 1)
                    def _():
                        wait_out(h - 1, o)

                    fire_gather(o)

                wait_gather(s)
                fire_out(h, s)

        wait_out(hist - 2, 0)
        wait_out(hist - 1, 1)

    return gather_kernel


@jax.jit
def kernel(x, table):
    batch, hist = x.shape
    vocab, dim = table.shape
    xt = x.T
    out_hm = _make_gather(vocab, dim, hist, batch)(table, xt)
    return jnp.transpose(out_hm, (1, 0, 2))
